# baseline (speedup 1.0000x reference)
import functools

import jax
import jax.numpy as jnp
from jax import lax
from jax.experimental import pallas as pl
from jax.experimental.pallas import tpu as pltpu
from jax.experimental.pallas import tpu_sc as plsc

EMB = 1000
D = 128
B = 16384
L = 16
NC, NS = 2, 16
NW = NC * NS
PW = B // NW
CHUNK = 128
NCHUNK = PW // CHUNK

LN2 = 0.6931471805599453
C75 = 3.4538776394910684


def _ln(x):
    bits = plsc.bitcast(x, jnp.int32)
    e = ((bits >> 23) & 0xFF) - 127
    m = plsc.bitcast((bits & 0x7FFFFF) | 0x3F800000, jnp.float32)
    big = m > 1.3333334
    m = jnp.where(big, m * 0.5, m)
    e = (e + jnp.where(big, 1, 0)).astype(jnp.float32)
    z = (m - 1.0) / (m + 1.0)
    z2 = z * z
    lnm = 2.0 * z * (1.0 + z2 * (0.33333334 + z2 * (0.2 + z2 * 0.14285715)))
    return e * LN2 + lnm


def _glove_body(widx_h, cidx_h, wemb_h, cemh_h, wb_h, cb_h, cooc_h, out_h,
                widx_v, cidx_v, flat_v, wbt_v, cbt_v, cc_v, s_v, wf_v,
                wrow_v, crow_v, acc_v,
                sem_wb, sem_cb, sem_cc, sem_w0, sem_w1, sem_c0, sem_c1):
    cemb_h = cemh_h
    c = lax.axis_index("c")
    s = lax.axis_index("s")
    wid = s * NC + c
    base = wid * PW
    sem_w = (sem_w0, sem_w1)
    sem_c = (sem_c0, sem_c1)

    pltpu.sync_copy(widx_h.at[pl.ds(base, PW)], widx_v)
    pltpu.sync_copy(cidx_h.at[pl.ds(base, PW)], cidx_v)

    cp_wb = pltpu.async_copy(wb_h, wbt_v, sem_wb)
    cp_cb = pltpu.async_copy(cb_h, cbt_v, sem_cb)

    cps = [None, None]

    def fire(k):
        b = k % 2
        cw = pltpu.async_copy(
            wemb_h.at[widx_v.at[pl.ds(k * CHUNK, CHUNK)]], wrow_v.at[b],
            sem_w[b])
        cx = pltpu.async_copy(
            cemb_h.at[cidx_v.at[pl.ds(k * CHUNK, CHUNK)]], crow_v.at[b],
            sem_c[b])
        cps[b] = (cw, cx)

    fire(0)
    fire(1)

    def flat_body(i, _):
        o = i * L
        flat_v[pl.ds(o, L)] = widx_v[pl.ds(o, L)] * EMB + cidx_v[pl.ds(o, L)]
        return 0

    lax.fori_loop(0, PW // L, flat_body, 0)
    cp_cc = pltpu.async_copy(cooc_h.at[flat_v], cc_v, sem_cc)

    cp_wb.wait()
    cp_cb.wait()
    cp_cc.wait()

    def prep_body(i, _):
        o = i * L
        lncc = _ln(cc_v[pl.ds(o, L)])
        wf = jnp.minimum(jnp.exp(0.75 * lncc - C75), 1.0)
        wb = plsc.load_gather(wbt_v, [widx_v[pl.ds(o, L)]])
        cb = plsc.load_gather(cbt_v, [cidx_v[pl.ds(o, L)]])
        s_v[pl.ds(o, L)] = wb + cb - lncc
        wf_v[pl.ds(o, L)] = wf
        return 0

    lax.fori_loop(0, PW // L, prep_body, 0)

    acc = jnp.zeros((L,), jnp.float32)
    for k in range(NCHUNK):
        b = k % 2
        cw, cx = cps[b]
        cw.wait()
        cx.wait()
        wr = wrow_v.at[b]
        cr = crow_v.at[b]

        def pair_body(p, a, k=k, wr=wr, cr=cr):
            g = jnp.full((L,), k * CHUNK, jnp.int32) + p
            sv = plsc.load_gather(s_v, [g])
            wfv = plsc.load_gather(wf_v, [g])
            cs = []
            for j in range(D // (2 * L)):
                w01 = plsc.unpack(
                    plsc.bitcast(wr[p, pl.ds(j * L, L)], jnp.bfloat16),
                    format=plsc.PackFormat.INTERLEAVED)
                c01 = plsc.unpack(
                    plsc.bitcast(cr[p, pl.ds(j * L, L)], jnp.bfloat16),
                    format=plsc.PackFormat.INTERLEAVED)
                for w_, c_ in zip(w01, c01):
                    t = w_ * c_ + sv
                    cs.append((wfv * t) * t)
            while len(cs) > 1:
                cs = [cs[i] + cs[i + 1] for i in range(0, len(cs), 2)]
            return a + cs[0]

        acc = plsc.parallel_loop(0, CHUNK, unroll=2, carry=acc)(pair_body)
        if k + 2 < NCHUNK:
            fire(k + 2)

    acc_v[...] = acc
    pltpu.sync_copy(acc_v, out_h.at[wid])


@jax.jit
def _glove(w_idx, c_idx, w_emb, c_emb, wb, cb, cooc_flat):
    mesh = plsc.VectorSubcoreMesh(core_axis_name="c", subcore_axis_name="s")
    f = pl.kernel(
        _glove_body,
        out_type=jax.ShapeDtypeStruct((NW, L), jnp.float32),
        mesh=mesh,
        compiler_params=pltpu.CompilerParams(
            needs_layout_passes=False, use_tc_tiling_on_sc=False),
        scratch_types=[
            pltpu.VMEM((PW,), jnp.int32),
            pltpu.VMEM((PW,), jnp.int32),
            pltpu.VMEM((PW,), jnp.int32),
            pltpu.VMEM((EMB,), jnp.float32),
            pltpu.VMEM((EMB,), jnp.float32),
            pltpu.VMEM((PW,), jnp.float32),
            pltpu.VMEM((PW,), jnp.float32),
            pltpu.VMEM((PW,), jnp.float32),
            pltpu.VMEM((2, CHUNK, D // 2), jnp.int32),
            pltpu.VMEM((2, CHUNK, D // 2), jnp.int32),
            pltpu.VMEM((L,), jnp.float32),
            pltpu.SemaphoreType.DMA,
            pltpu.SemaphoreType.DMA,
            pltpu.SemaphoreType.DMA,
            pltpu.SemaphoreType.DMA,
            pltpu.SemaphoreType.DMA,
            pltpu.SemaphoreType.DMA,
            pltpu.SemaphoreType.DMA,
        ],
    )
    partials = f(w_idx, c_idx, w_emb, c_emb, wb, cb, cooc_flat)
    return jnp.sum(partials)


def kernel(w_idx, c_idx, w_emb, c_emb, w_bias, c_bias, cooc):
    return _glove(
        w_idx.astype(jnp.int32),
        c_idx.astype(jnp.int32),
        lax.bitcast_convert_type(
            w_emb.astype(jnp.bfloat16).reshape(EMB, D // 2, 2), jnp.int32),
        lax.bitcast_convert_type(
            c_emb.astype(jnp.bfloat16).reshape(EMB, D // 2, 2), jnp.int32),
        w_bias.reshape(EMB),
        c_bias.reshape(EMB),
        cooc.reshape(EMB * EMB),
    )


# speedup vs baseline: 1.1267x; 1.1267x over previous
import functools

import jax
import jax.numpy as jnp
from jax import lax
from jax.experimental import pallas as pl
from jax.experimental.pallas import tpu as pltpu
from jax.experimental.pallas import tpu_sc as plsc

EMB = 1000
D = 128
B = 16384
L = 16
NC, NS = 2, 16
NW = NC * NS
PW = B // NW
CHUNK = 128
NCHUNK = PW // CHUNK

LN2 = 0.6931471805599453
C75 = 3.4538776394910684


def _ln(x):
    bits = plsc.bitcast(x, jnp.int32)
    e = ((bits >> 23) & 0xFF) - 127
    m = plsc.bitcast((bits & 0x7FFFFF) | 0x3F800000, jnp.float32)
    big = m > 1.3333334
    m = jnp.where(big, m * 0.5, m)
    e = (e + jnp.where(big, 1, 0)).astype(jnp.float32)
    z = (m - 1.0) / (m + 1.0)
    z2 = z * z
    lnm = 2.0 * z * (1.0 + z2 * (0.33333334 + z2 * (0.2 + z2 * 0.14285715)))
    return e * LN2 + lnm


def _glove_body(widx_h, cidx_h, wemb_h, cemh_h, wb_h, cb_h, cooc_h, out_h,
                widx_v, cidx_v, flat_v, wbt_v, cbt_v, cc_v, s_v, wf_v,
                wrow_v, crow_v, acc_v, shw_v, shc_v,
                sem_wb, sem_cb, sem_cc, sem_w0, sem_w1, sem_c0, sem_c1):
    cemb_h = cemh_h
    c = lax.axis_index("c")
    s = lax.axis_index("s")
    wid = s * NC + c
    base = wid * PW
    sem_w = (sem_w0, sem_w1)
    sem_c = (sem_c0, sem_c1)

    cp_wb = pltpu.async_copy(wb_h, wbt_v, sem_wb)
    cp_cb = pltpu.async_copy(cb_h, cbt_v, sem_cb)

    @pl.when(s < 8)
    def _():
        o = s * 125
        pltpu.sync_copy(wemb_h.at[pl.ds(o, 125)], shw_v.at[pl.ds(o, 125)])

    @pl.when(s >= 8)
    def _():
        o = (s - 8) * 125
        pltpu.sync_copy(cemb_h.at[pl.ds(o, 125)], shc_v.at[pl.ds(o, 125)])

    pltpu.sync_copy(widx_h.at[pl.ds(base, PW)], widx_v)
    pltpu.sync_copy(cidx_h.at[pl.ds(base, PW)], cidx_v)

    cps = [None, None]

    def fire(k):
        b = k % 2
        cw = pltpu.async_copy(
            shw_v.at[widx_v.at[pl.ds(k * CHUNK, CHUNK)]], wrow_v.at[b],
            sem_w[b])
        cx = pltpu.async_copy(
            shc_v.at[cidx_v.at[pl.ds(k * CHUNK, CHUNK)]], crow_v.at[b],
            sem_c[b])
        cps[b] = (cw, cx)

    def flat_body(i, _):
        o = i * L
        flat_v[pl.ds(o, L)] = widx_v[pl.ds(o, L)] * EMB + cidx_v[pl.ds(o, L)]
        return 0

    lax.fori_loop(0, PW // L, flat_body, 0)
    cp_cc = pltpu.async_copy(cooc_h.at[flat_v], cc_v, sem_cc)

    cp_wb.wait()
    cp_cb.wait()
    cp_cc.wait()

    def prep_body(i, _):
        o = i * L
        lncc = _ln(cc_v[pl.ds(o, L)])
        wf = jnp.minimum(jnp.exp(0.75 * lncc - C75), 1.0)
        wb = plsc.load_gather(wbt_v, [widx_v[pl.ds(o, L)]])
        cb = plsc.load_gather(cbt_v, [cidx_v[pl.ds(o, L)]])
        s_v[pl.ds(o, L)] = wb + cb - lncc
        wf_v[pl.ds(o, L)] = wf
        return 0

    lax.fori_loop(0, PW // L, prep_body, 0)

    plsc.subcore_barrier()
    fire(0)
    fire(1)

    acc = jnp.zeros((L,), jnp.float32)
    for k in range(NCHUNK):
        b = k % 2
        cw, cx = cps[b]
        cw.wait()
        cx.wait()
        wr = wrow_v.at[b]
        cr = crow_v.at[b]

        def pair_body(p, a, k=k, wr=wr, cr=cr):
            g = jnp.full((L,), k * CHUNK, jnp.int32) + p
            sv = plsc.load_gather(s_v, [g])
            wfv = plsc.load_gather(wf_v, [g])
            cs = []
            for j in range(D // L):
                t = wr[p, pl.ds(j * L, L)] * cr[p, pl.ds(j * L, L)] + sv
                cs.append((wfv * t) * t)
            while len(cs) > 1:
                cs = [cs[i] + cs[i + 1] for i in range(0, len(cs), 2)]
            return a + cs[0]

        acc = plsc.parallel_loop(0, CHUNK, unroll=2, carry=acc)(pair_body)
        if k + 2 < NCHUNK:
            fire(k + 2)

    acc_v[...] = acc
    pltpu.sync_copy(acc_v, out_h.at[wid])


@jax.jit
def _glove(w_idx, c_idx, w_emb, c_emb, wb, cb, cooc_flat):
    mesh = plsc.VectorSubcoreMesh(core_axis_name="c", subcore_axis_name="s")
    f = pl.kernel(
        _glove_body,
        out_type=jax.ShapeDtypeStruct((NW, L), jnp.float32),
        mesh=mesh,
        compiler_params=pltpu.CompilerParams(
            needs_layout_passes=False, use_tc_tiling_on_sc=False),
        scratch_types=[
            pltpu.VMEM((PW,), jnp.int32),
            pltpu.VMEM((PW,), jnp.int32),
            pltpu.VMEM((PW,), jnp.int32),
            pltpu.VMEM((EMB,), jnp.float32),
            pltpu.VMEM((EMB,), jnp.float32),
            pltpu.VMEM((PW,), jnp.float32),
            pltpu.VMEM((PW,), jnp.float32),
            pltpu.VMEM((PW,), jnp.float32),
            pltpu.VMEM((2, CHUNK, D), jnp.float32),
            pltpu.VMEM((2, CHUNK, D), jnp.float32),
            pltpu.VMEM((L,), jnp.float32),
            pltpu.VMEM_SHARED((EMB, D), jnp.float32),
            pltpu.VMEM_SHARED((EMB, D), jnp.float32),
            pltpu.SemaphoreType.DMA,
            pltpu.SemaphoreType.DMA,
            pltpu.SemaphoreType.DMA,
            pltpu.SemaphoreType.DMA,
            pltpu.SemaphoreType.DMA,
            pltpu.SemaphoreType.DMA,
            pltpu.SemaphoreType.DMA,
        ],
    )
    partials = f(w_idx, c_idx, w_emb, c_emb, wb, cb, cooc_flat)
    return jnp.sum(partials)


def kernel(w_idx, c_idx, w_emb, c_emb, w_bias, c_bias, cooc):
    return _glove(
        w_idx.astype(jnp.int32),
        c_idx.astype(jnp.int32),
        w_emb,
        c_emb,
        w_bias.reshape(EMB),
        c_bias.reshape(EMB),
        cooc.reshape(EMB * EMB),
    )


# speedup vs baseline: 1.1723x; 1.0404x over previous
import functools

import jax
import jax.numpy as jnp
from jax import lax
from jax.experimental import pallas as pl
from jax.experimental.pallas import tpu as pltpu
from jax.experimental.pallas import tpu_sc as plsc

EMB = 1000
D = 128
B = 16384
L = 16
NC, NS = 2, 16
NW = NC * NS
PW = B // NW
CHUNK = 128
NCHUNK = PW // CHUNK

LN2 = 0.6931471805599453
C75 = 3.4538776394910684


def _ln(x):
    bits = plsc.bitcast(x, jnp.int32)
    e = ((bits >> 23) & 0xFF) - 127
    m = plsc.bitcast((bits & 0x7FFFFF) | 0x3F800000, jnp.float32)
    big = m > 1.3333334
    m = jnp.where(big, m * 0.5, m)
    e = (e + jnp.where(big, 1, 0)).astype(jnp.float32)
    z = (m - 1.0) / (m + 1.0)
    z2 = z * z
    lnm = 2.0 * z * (1.0 + z2 * (0.33333334 + z2 * (0.2 + z2 * 0.14285715)))
    return e * LN2 + lnm


def _glove_body(widx_h, cidx_h, wemb_h, cemh_h, wb_h, cb_h, cooc_h, out_h,
                widx_v, cidx_v, flat_v, wbt_v, cbt_v, cc_v, s_v, wf_v,
                wrow_v, crow_v, acc_v, shw_v, shc_v,
                sem_wb, sem_cb, sem_cc, sem_w0, sem_w1, sem_c0, sem_c1):
    cemb_h = cemh_h
    c = lax.axis_index("c")
    s = lax.axis_index("s")
    wid = s * NC + c
    base = wid * PW
    sem_w = (sem_w0, sem_w1)
    sem_c = (sem_c0, sem_c1)

    cp_wb = pltpu.async_copy(wb_h, wbt_v, sem_wb)
    cp_cb = pltpu.async_copy(cb_h, cbt_v, sem_cb)

    @pl.when(s < 8)
    def _():
        o = s * 125
        pltpu.sync_copy(wemb_h.at[pl.ds(o, 125)], shw_v.at[pl.ds(o, 125)])

    @pl.when(s >= 8)
    def _():
        o = (s - 8) * 125
        pltpu.sync_copy(cemb_h.at[pl.ds(o, 125)], shc_v.at[pl.ds(o, 125)])

    pltpu.sync_copy(widx_h.at[pl.ds(base, PW)], widx_v)
    pltpu.sync_copy(cidx_h.at[pl.ds(base, PW)], cidx_v)

    cps = [None, None]

    def fire(k):
        b = k % 2
        cw = pltpu.async_copy(
            shw_v.at[widx_v.at[pl.ds(k * CHUNK, CHUNK)]], wrow_v.at[b],
            sem_w[b])
        cx = pltpu.async_copy(
            shc_v.at[cidx_v.at[pl.ds(k * CHUNK, CHUNK)]], crow_v.at[b],
            sem_c[b])
        cps[b] = (cw, cx)

    def flat_body(i, _):
        o = i * L
        flat_v[pl.ds(o, L)] = widx_v[pl.ds(o, L)] * EMB + cidx_v[pl.ds(o, L)]
        return 0

    lax.fori_loop(0, PW // L, flat_body, 0)
    cp_cc = pltpu.async_copy(cooc_h.at[flat_v], cc_v, sem_cc)

    plsc.subcore_barrier()
    fire(0)
    fire(1)

    cp_wb.wait()
    cp_cb.wait()
    cp_cc.wait()

    def prep_body(i, _):
        o = i * L
        lncc = _ln(cc_v[pl.ds(o, L)])
        wf = jnp.minimum(jnp.exp(0.75 * lncc - C75), 1.0)
        wb = plsc.load_gather(wbt_v, [widx_v[pl.ds(o, L)]])
        cb = plsc.load_gather(cbt_v, [cidx_v[pl.ds(o, L)]])
        s_v[pl.ds(o, L)] = wb + cb - lncc
        wf_v[pl.ds(o, L)] = wf
        return 0

    lax.fori_loop(0, PW // L, prep_body, 0)

    acc = jnp.zeros((L,), jnp.float32)
    for k in range(NCHUNK):
        b = k % 2
        cw, cx = cps[b]
        cw.wait()
        cx.wait()
        wr = wrow_v.at[b]
        cr = crow_v.at[b]

        def pair_body(p, a, k=k, wr=wr, cr=cr):
            g = jnp.full((L,), k * CHUNK, jnp.int32) + p
            sv = plsc.load_gather(s_v, [g])
            wfv = plsc.load_gather(wf_v, [g])
            cs = []
            for j in range(D // L):
                t = wr[p, pl.ds(j * L, L)] * cr[p, pl.ds(j * L, L)] + sv
                cs.append((wfv * t) * t)
            while len(cs) > 1:
                cs = [cs[i] + cs[i + 1] for i in range(0, len(cs), 2)]
            return a + cs[0]

        acc = plsc.parallel_loop(0, CHUNK, unroll=2, carry=acc)(pair_body)
        if k + 2 < NCHUNK:
            fire(k + 2)

    acc_v[...] = acc
    pltpu.sync_copy(acc_v, out_h.at[wid])


@jax.jit
def _glove(w_idx, c_idx, w_emb, c_emb, wb, cb, cooc_flat):
    mesh = plsc.VectorSubcoreMesh(core_axis_name="c", subcore_axis_name="s")
    f = pl.kernel(
        _glove_body,
        out_type=jax.ShapeDtypeStruct((NW, L), jnp.float32),
        mesh=mesh,
        compiler_params=pltpu.CompilerParams(
            needs_layout_passes=False, use_tc_tiling_on_sc=False),
        scratch_types=[
            pltpu.VMEM((PW,), jnp.int32),
            pltpu.VMEM((PW,), jnp.int32),
            pltpu.VMEM((PW,), jnp.int32),
            pltpu.VMEM((EMB,), jnp.float32),
            pltpu.VMEM((EMB,), jnp.float32),
            pltpu.VMEM((PW,), jnp.float32),
            pltpu.VMEM((PW,), jnp.float32),
            pltpu.VMEM((PW,), jnp.float32),
            pltpu.VMEM((2, CHUNK, D), jnp.float32),
            pltpu.VMEM((2, CHUNK, D), jnp.float32),
            pltpu.VMEM((L,), jnp.float32),
            pltpu.VMEM_SHARED((EMB, D), jnp.float32),
            pltpu.VMEM_SHARED((EMB, D), jnp.float32),
            pltpu.SemaphoreType.DMA,
            pltpu.SemaphoreType.DMA,
            pltpu.SemaphoreType.DMA,
            pltpu.SemaphoreType.DMA,
            pltpu.SemaphoreType.DMA,
            pltpu.SemaphoreType.DMA,
            pltpu.SemaphoreType.DMA,
        ],
    )
    partials = f(w_idx, c_idx, w_emb, c_emb, wb, cb, cooc_flat)
    return jnp.sum(partials)


def kernel(w_idx, c_idx, w_emb, c_emb, w_bias, c_bias, cooc):
    return _glove(
        w_idx.astype(jnp.int32),
        c_idx.astype(jnp.int32),
        w_emb,
        c_emb,
        w_bias.reshape(EMB),
        c_bias.reshape(EMB),
        cooc.reshape(EMB * EMB),
    )
